# baseline (speedup 1.0000x reference)
import functools

import jax
import jax.numpy as jnp
from jax import lax
from jax.experimental import pallas as pl
from jax.experimental.pallas import tpu as pltpu
from jax.experimental.pallas import tpu_sc as plsc

N = 10000
FEAT = 1040
H = 16
C = 65
E = 76800

NC = 2
NS = 16
EP = 98304
TPE = EP // (NC * NS)
NB = TPE // 128
SEP = EP // NS
SGRP = SEP // 16
DST_SENTINEL = 16000
CHUNK = 512
CSHIFT = 9
NCHUNKS = 20
ROWS_PER_TILE = CHUNK // NS
BSZ = 32
MBLK = 400
GRID = N // MBLK

_f32 = jnp.float32
_i32 = jnp.int32


def _proj_body(x_ref, w_ref, ss_ref, sd_ref, h_ref, as_ref, ad_ref,
               ms_ref, md_ref):
    i = pl.program_id(0)
    h = jnp.dot(x_ref[...], w_ref[...], preferred_element_type=_f32)
    h_ref[...] = h
    a_s = jnp.dot(h, ss_ref[...], preferred_element_type=_f32)
    a_d = jnp.dot(h, sd_ref[...], preferred_element_type=_f32)
    as_ref[...] = a_s
    ad_ref[...] = a_d
    bms = jnp.max(a_s)
    bmd = jnp.max(a_d)

    @pl.when(i == 0)
    def _():
        ms_ref[0, 0] = bms
        md_ref[0, 0] = bmd

    @pl.when(i > 0)
    def _():
        ms_ref[0, 0] = jnp.maximum(ms_ref[0, 0], bms)
        md_ref[0, 0] = jnp.maximum(md_ref[0, 0], bmd)


def _project(x, W, S_src, S_dst):
    return pl.pallas_call(
        _proj_body,
        grid=(GRID,),
        in_specs=[
            pl.BlockSpec((MBLK, FEAT), lambda i: (i, 0)),
            pl.BlockSpec((FEAT, FEAT), lambda i: (0, 0)),
            pl.BlockSpec((FEAT, H), lambda i: (0, 0)),
            pl.BlockSpec((FEAT, H), lambda i: (0, 0)),
        ],
        out_specs=[
            pl.BlockSpec((MBLK, FEAT), lambda i: (i, 0)),
            pl.BlockSpec((MBLK, H), lambda i: (i, 0)),
            pl.BlockSpec((MBLK, H), lambda i: (i, 0)),
            pl.BlockSpec((1, 1), lambda i: (0, 0), memory_space=pltpu.SMEM),
            pl.BlockSpec((1, 1), lambda i: (0, 0), memory_space=pltpu.SMEM),
        ],
        out_shape=[
            jax.ShapeDtypeStruct((N, FEAT), _f32),
            jax.ShapeDtypeStruct((N, H), _f32),
            jax.ShapeDtypeStruct((N, H), _f32),
            jax.ShapeDtypeStruct((1, 1), _f32),
            jax.ShapeDtypeStruct((1, 1), _f32),
        ],
    )(x, W, S_src, S_dst)


def _b1_body(src2, dst2, a_s, a_d, m16, w_out, spart_out,
             s_acc, src_v, dst_v, as_r, ad_r, w_r, m_v, z_r):
    c = lax.axis_index("c")
    s = lax.axis_index("s")
    wid = s * NC + c
    pltpu.sync_copy(src2.at[pl.ds(wid * NB, NB)], src_v)
    pltpu.sync_copy(dst2.at[pl.ds(wid * NB, NB)], dst_v)
    pltpu.sync_copy(m16, m_v)
    mv = m_v[...]

    def _zb(j, _):
        z_r[j] = jnp.zeros((16,), _f32)
        return 0
    lax.fori_loop(0, 200, _zb, 0)

    @pl.when(s < 10)
    def _():
        for q in range(5):
            pltpu.sync_copy(z_r, s_acc.at[pl.ds(s * 1000 + q * 200, 200)])
    plsc.subcore_barrier()

    for b in range(NB):
        gbase = wid * TPE + b * 128
        pltpu.sync_copy(a_s.at[src_v.at[b]], as_r)
        pltpu.sync_copy(a_d.at[dst_v.at[b]], ad_r)

        def _wb(j, _):
            e = as_r[j] + ad_r[j]
            e = jnp.where(e >= 0.0, e, 0.2 * e)
            wv = jnp.exp(e - mv)
            valid = jnp.where(gbase + j < E + N, 1.0, 0.0).astype(_f32)
            w_r[j] = wv * valid
            return 0
        lax.fori_loop(0, 128, _wb, 0)
        pltpu.sync_copy(w_r, w_out.at[pl.ds(gbase, 128)])
        pltpu.sync_copy(w_r, s_acc.at[dst_v.at[b]], add=True)

    plsc.subcore_barrier()

    @pl.when(s < 10)
    def _():
        for q in range(5):
            r0 = s * 1000 + q * 200
            pltpu.sync_copy(s_acc.at[pl.ds(r0, 200)],
                            spart_out.at[c, pl.ds(r0, 200)])


def _b1(src2, dst2, a_s, a_d, m16):
    f = pl.kernel(
        _b1_body,
        out_type=[
            jax.ShapeDtypeStruct((EP, H), _f32),
            jax.ShapeDtypeStruct((NC, N, H), _f32),
        ],
        mesh=plsc.VectorSubcoreMesh(core_axis_name="c", subcore_axis_name="s"),
        compiler_params=pltpu.CompilerParams(
            use_tc_tiling_on_sc=False, needs_layout_passes=False),
        scratch_types=[
            pltpu.VMEM_SHARED((N, H), _f32),
            pltpu.VMEM((NB, 128), _i32),
            pltpu.VMEM((NB, 128), _i32),
            pltpu.VMEM((128, H), _f32),
            pltpu.VMEM((128, H), _f32),
            pltpu.VMEM((128, H), _f32),
            pltpu.VMEM((16,), _f32),
            pltpu.VMEM((200, H), _f32),
        ],
    )
    return f(src2, dst2, a_s, a_d, m16)


def _b2_body(dst2, w_in, s0, s1, alpha_out,
             dst_v, w_r, s0_r, s1_r, a_r):
    c = lax.axis_index("c")
    s = lax.axis_index("s")
    wid = s * NC + c
    pltpu.sync_copy(dst2.at[pl.ds(wid * NB, NB)], dst_v)
    for b in range(NB):
        gbase = wid * TPE + b * 128
        pltpu.sync_copy(w_in.at[pl.ds(gbase, 128)], w_r)
        pltpu.sync_copy(s0.at[dst_v.at[b]], s0_r)
        pltpu.sync_copy(s1.at[dst_v.at[b]], s1_r)

        def _ab(j, _):
            a_r[j] = w_r[j] / (s0_r[j] + s1_r[j] + 1e-16)
            return 0
        lax.fori_loop(0, 128, _ab, 0)
        pltpu.sync_copy(a_r, alpha_out.at[pl.ds(gbase, 128)])


def _b2(dst2, w, s0, s1):
    f = pl.kernel(
        _b2_body,
        out_type=[jax.ShapeDtypeStruct((EP, H), _f32)],
        mesh=plsc.VectorSubcoreMesh(core_axis_name="c", subcore_axis_name="s"),
        compiler_params=pltpu.CompilerParams(
            use_tc_tiling_on_sc=False, needs_layout_passes=False),
        scratch_types=[
            pltpu.VMEM((NB, 128), _i32),
            pltpu.VMEM((128, H), _f32),
            pltpu.VMEM((128, H), _f32),
            pltpu.VMEM((128, H), _f32),
            pltpu.VMEM((128, H), _f32),
        ],
    )
    return f(dst2, w, s0, s1)[0]


def _b3_body(src2, dst2, alpha, h, hidx, att_out,
             acc, src_v, dst_v, ids_v, hrow2, arow2, zrow, hidx_v,
             nidx2, didx2, eidx2, gsem, asem, ssem):
    c = lax.axis_index("c")
    s = lax.axis_index("s")
    sbase = s * SEP
    pltpu.sync_copy(src2.at[pl.ds(s * (SEP // 128), SEP // 128)], src_v)
    pltpu.sync_copy(dst2.at[pl.ds(s * (SEP // 128), SEP // 128)], dst_v)
    pltpu.sync_copy(hidx, hidx_v)

    def _zj(j, _):
        def _zk(k, _):
            zrow[j, pl.ds(k * 16, 16)] = jnp.zeros((16,), _f32)
            return 0
        lax.fori_loop(0, C, _zk, 0)
        return 0
    lax.fori_loop(0, 8, _zj, 0)

    iota16 = lax.iota(_i32, 16)
    gdn = lax.GatherDimensionNumbers(
        offset_dims=(), collapsed_slice_dims=(0,), start_index_map=(0,))

    def _vgather(x, idx):
        return lax.gather(x, idx[:, None], gdn, (1,),
                          mode=lax.GatherScatterMode.PROMISE_IN_BOUNDS)

    shifts = [(jnp.maximum(iota16 - stp, 0), iota16 >= stp)
              for stp in (1, 2, 4, 8)]

    def _prefix_inclusive(v):
        for idx, sel in shifts:
            sh = jnp.where(sel, _vgather(v, idx), 0)
            v = v + sh
        return v

    def _zi(g, _):
        ids_v[pl.ds(g * 16, 16)] = jnp.zeros((16,), _i32)
        return 0
    lax.fori_loop(0, SGRP + 1, _zi, 0)

    def _chunk_body(chunk, _):
        cid = chunk * NC + c
        base = cid * CHUNK

        def _za(q, _):
            pltpu.sync_copy(zrow,
                            acc.at[pl.ds(s * ROWS_PER_TILE + q * 8, 8)])
            return 0
        lax.fori_loop(0, ROWS_PER_TILE // 8, _za, 0)
        plsc.subcore_barrier()

        def _scan(g, off):
            dv = dst_v[g >> 3, pl.ds((g & 7) * 16, 16)]
            mask = lax.shift_right_logical(dv, CSHIFT) == cid
            ids = iota16 + g * 16
            cnt = _prefix_inclusive(mask.astype(_i32))
            pos = off + cnt - 1
            plsc.store_scatter(ids_v, [pos], ids, mask=mask)
            return off + plsc.all_reduce_population_count(mask)[0]
        m = lax.fori_loop(0, SGRP, _scan, 0)

        nb = (m + BSZ - 1) // BSZ

        def _prep(bi, slot):
            for half in range(BSZ // 16):
                idsv = ids_v[pl.ds(bi * BSZ + half * 16, 16)]
                gr = lax.shift_right_logical(idsv, 7)
                gc = jnp.bitwise_and(idsv, 127)
                sv = plsc.load_gather(src_v, [gr, gc])
                dv = plsc.load_gather(dst_v, [gr, gc])
                nidx2[slot, pl.ds(half * 16, 16)] = sv
                didx2[slot, pl.ds(half * 16, 16)] = jnp.clip(
                    dv - base, 0, CHUNK - 1)
                eidx2[slot, pl.ds(half * 16, 16)] = idsv + sbase
            pltpu.async_copy(h.at[nidx2.at[slot]], hrow2.at[slot], gsem)
            pltpu.async_copy(alpha.at[eidx2.at[slot]], arow2.at[slot], asem)

        @pl.when(nb > 0)
        def _():
            _prep(0, 0)

        def _batch(b, _):
            sl = jnp.bitwise_and(b, 1)
            nsl = 1 - sl

            @pl.when(jnp.logical_and(b >= 1, b + 1 < nb))
            def _():
                pltpu.make_async_copy(h.at[nidx2.at[nsl]],
                                      hrow2.at[nsl], ssem).wait()

            @pl.when(b + 1 < nb)
            def _():
                _prep(b + 1, nsl)

            pltpu.make_async_copy(h.at[nidx2.at[sl]],
                                  hrow2.at[sl], gsem).wait()
            pltpu.make_async_copy(alpha.at[eidx2.at[sl]],
                                  arow2.at[sl], asem).wait()

            for grp in range(BSZ // 16):
                avs = [arow2[sl, grp * 16 + j] *
                       jnp.where(b * BSZ + grp * 16 + j < m,
                                 1.0, 0.0).astype(_f32)
                       for j in range(16)]

                def _kb(k, _, _avs=avs, _g=grp):
                    hx = hidx_v[pl.ds(k * 16, 16)]
                    for j in range(16):
                        hv = hrow2[sl, _g * 16 + j, pl.ds(k * 16, 16)]
                        hrow2[sl, _g * 16 + j, pl.ds(k * 16, 16)] = (
                            hv * _vgather(_avs[j], hx))
                    return 0
                lax.fori_loop(0, C, _kb, 0)
            pltpu.async_copy(hrow2.at[sl], acc.at[didx2.at[sl]], ssem,
                             add=True)
            return 0
        lax.fori_loop(0, nb, _batch, 0)

        def _drain(d, _):
            pltpu.make_async_copy(h.at[nidx2.at[0]], hrow2.at[0], ssem).wait()
            return 0
        lax.fori_loop(0, jnp.minimum(nb, 2), _drain, 0)
        plsc.subcore_barrier()

        pltpu.sync_copy(acc.at[pl.ds(s * ROWS_PER_TILE, ROWS_PER_TILE)],
                        att_out.at[pl.ds(base + s * ROWS_PER_TILE,
                                         ROWS_PER_TILE)])
        plsc.subcore_barrier()
        return 0

    lax.fori_loop(0, NCHUNKS // NC, _chunk_body, 0)


def _b3(src2, dstb2, alpha, h, hidx):
    f = pl.kernel(
        _b3_body,
        out_type=[jax.ShapeDtypeStruct((NCHUNKS * CHUNK, FEAT), _f32)],
        mesh=plsc.VectorSubcoreMesh(core_axis_name="c", subcore_axis_name="s"),
        compiler_params=pltpu.CompilerParams(
            use_tc_tiling_on_sc=False, needs_layout_passes=False),
        scratch_types=[
            pltpu.VMEM_SHARED((CHUNK, FEAT), _f32),
            pltpu.VMEM((SEP // 128, 128), _i32),
            pltpu.VMEM((SEP // 128, 128), _i32),
            pltpu.VMEM((SEP + 16,), _i32),
            pltpu.VMEM((2, BSZ, FEAT), _f32),
            pltpu.VMEM((2, BSZ, H), _f32),
            pltpu.VMEM((8, FEAT), _f32),
            pltpu.VMEM((FEAT,), _i32),
            pltpu.VMEM((2, BSZ), _i32),
            pltpu.VMEM((2, BSZ), _i32),
            pltpu.VMEM((2, BSZ), _i32),
            pltpu.SemaphoreType.DMA,
            pltpu.SemaphoreType.DMA,
            pltpu.SemaphoreType.DMA,
        ],
    )
    return f(src2, dstb2, alpha, h, hidx)[0]


def _c1_body(prev_ref, att_ref, bias_ref, s1_ref, s2_ref):
    i = pl.program_id(0)
    z = prev_ref[...] + att_ref[...] + bias_ref[...]
    s1 = jnp.sum(z, axis=0, keepdims=True)
    s2 = jnp.sum(z * z, axis=0, keepdims=True)

    @pl.when(i == 0)
    def _():
        s1_ref[...] = s1
        s2_ref[...] = s2

    @pl.when(i > 0)
    def _():
        s1_ref[...] += s1
        s2_ref[...] += s2


def _c2_body(prev_ref, att_ref, bias_ref, gamma_ref, beta_ref,
             s1_ref, s2_ref, y_ref):
    z = prev_ref[...] + att_ref[...] + bias_ref[...]
    mean = s1_ref[...] / N
    var = s2_ref[...] / N - mean * mean
    inv = lax.rsqrt(var + 1e-5)
    y = gamma_ref[...] * ((z - mean) * inv) + beta_ref[...]
    y_ref[...] = jnp.maximum(y, 0.0)


def _bn(prev, att, bias2, gamma2, beta2):
    s1, s2 = pl.pallas_call(
        _c1_body,
        grid=(GRID,),
        in_specs=[
            pl.BlockSpec((MBLK, FEAT), lambda i: (i, 0)),
            pl.BlockSpec((MBLK, FEAT), lambda i: (i, 0)),
            pl.BlockSpec((1, FEAT), lambda i: (0, 0)),
        ],
        out_specs=[
            pl.BlockSpec((1, FEAT), lambda i: (0, 0)),
            pl.BlockSpec((1, FEAT), lambda i: (0, 0)),
        ],
        out_shape=[
            jax.ShapeDtypeStruct((1, FEAT), _f32),
            jax.ShapeDtypeStruct((1, FEAT), _f32),
        ],
    )(prev, att, bias2)
    return pl.pallas_call(
        _c2_body,
        grid=(GRID,),
        in_specs=[
            pl.BlockSpec((MBLK, FEAT), lambda i: (i, 0)),
            pl.BlockSpec((MBLK, FEAT), lambda i: (i, 0)),
            pl.BlockSpec((1, FEAT), lambda i: (0, 0)),
            pl.BlockSpec((1, FEAT), lambda i: (0, 0)),
            pl.BlockSpec((1, FEAT), lambda i: (0, 0)),
            pl.BlockSpec((1, FEAT), lambda i: (0, 0)),
            pl.BlockSpec((1, FEAT), lambda i: (0, 0)),
        ],
        out_specs=pl.BlockSpec((MBLK, FEAT), lambda i: (i, 0)),
        out_shape=jax.ShapeDtypeStruct((N, FEAT), _f32),
    )(prev, att, bias2, gamma2, beta2, s1, s2)


def kernel(prev, x, edge_index, W, att_src, att_dst, bias, gamma, beta):
    loops = jnp.arange(N, dtype=_i32)
    pad = jnp.zeros((EP - E - N,), dtype=_i32)
    src = jnp.concatenate([edge_index[0], loops, pad])
    dst = jnp.concatenate([edge_index[1], loops, pad])
    dstb = jnp.concatenate(
        [edge_index[1], loops, jnp.full((EP - E - N,), DST_SENTINEL, _i32)])
    src2 = src.reshape(EP // 128, 128)
    dst2 = dst.reshape(EP // 128, 128)
    dstb2 = dstb.reshape(EP // 128, 128)

    rows = jnp.arange(FEAT)
    S_src = jnp.zeros((FEAT, H), _f32).at[rows, rows // C].set(
        att_src.reshape(-1))
    S_dst = jnp.zeros((FEAT, H), _f32).at[rows, rows // C].set(
        att_dst.reshape(-1))

    h, a_s, a_d, ms, md = _project(x, W, S_src, S_dst)

    v = ms[0, 0] + md[0, 0]
    M = jnp.where(v >= 0.0, v, 0.2 * v)
    m16 = jnp.full((16,), M, _f32)

    w, spart = _b1(src2, dst2, a_s, a_d, m16)
    alpha = _b2(dst2, w, spart[0], spart[1])

    hidx = (rows // C).astype(_i32)
    att = _b3(src2, dstb2, alpha, h, hidx)

    bias2 = bias.reshape(1, FEAT)
    gamma2 = gamma.reshape(1, FEAT)
    beta2 = beta.reshape(1, FEAT)
    return _bn(prev, att, bias2, gamma2, beta2)


# speedup vs baseline: 2.3547x; 2.3547x over previous
import functools

import jax
import jax.numpy as jnp
from jax import lax
from jax.experimental import pallas as pl
from jax.experimental.pallas import tpu as pltpu
from jax.experimental.pallas import tpu_sc as plsc

N = 10000
FEAT = 1040
H = 16
C = 65
E = 76800

NC = 2
NS = 16
EP = 98304
TPE = EP // (NC * NS)
NB = TPE // 128
SEP = EP // NS
SGRP = SEP // 16
DST_SENTINEL = 16000
CHUNK = 512
CSHIFT = 9
NCHUNKS = 20
ROWS_PER_TILE = CHUNK // NS
BSZ = 32
MBLK = 400
GRID = N // MBLK

_f32 = jnp.float32
_i32 = jnp.int32


def _proj_body(x_ref, w_ref, ss_ref, sd_ref, h_ref, as_ref, ad_ref,
               ms_ref, md_ref):
    i = pl.program_id(0)
    h = jnp.dot(x_ref[...], w_ref[...], preferred_element_type=_f32)
    h_ref[...] = h
    a_s = jnp.dot(h, ss_ref[...], preferred_element_type=_f32)
    a_d = jnp.dot(h, sd_ref[...], preferred_element_type=_f32)
    as_ref[...] = a_s
    ad_ref[...] = a_d
    bms = jnp.max(a_s)
    bmd = jnp.max(a_d)

    @pl.when(i == 0)
    def _():
        ms_ref[0, 0] = bms
        md_ref[0, 0] = bmd

    @pl.when(i > 0)
    def _():
        ms_ref[0, 0] = jnp.maximum(ms_ref[0, 0], bms)
        md_ref[0, 0] = jnp.maximum(md_ref[0, 0], bmd)


def _project(x, W, S_src, S_dst):
    return pl.pallas_call(
        _proj_body,
        grid=(GRID,),
        in_specs=[
            pl.BlockSpec((MBLK, FEAT), lambda i: (i, 0)),
            pl.BlockSpec((FEAT, FEAT), lambda i: (0, 0)),
            pl.BlockSpec((FEAT, H), lambda i: (0, 0)),
            pl.BlockSpec((FEAT, H), lambda i: (0, 0)),
        ],
        out_specs=[
            pl.BlockSpec((MBLK, FEAT), lambda i: (i, 0)),
            pl.BlockSpec((MBLK, H), lambda i: (i, 0)),
            pl.BlockSpec((MBLK, H), lambda i: (i, 0)),
            pl.BlockSpec((1, 1), lambda i: (0, 0), memory_space=pltpu.SMEM),
            pl.BlockSpec((1, 1), lambda i: (0, 0), memory_space=pltpu.SMEM),
        ],
        out_shape=[
            jax.ShapeDtypeStruct((N, FEAT), _f32),
            jax.ShapeDtypeStruct((N, H), _f32),
            jax.ShapeDtypeStruct((N, H), _f32),
            jax.ShapeDtypeStruct((1, 1), _f32),
            jax.ShapeDtypeStruct((1, 1), _f32),
        ],
    )(x, W, S_src, S_dst)


def _b1_body(src2, dst2, a_s, a_d, m16, w_out, spart_out,
             s_acc, src_v, dst_v, as_r, ad_r, w_r, m_v, z_r):
    c = lax.axis_index("c")
    s = lax.axis_index("s")
    wid = s * NC + c
    pltpu.sync_copy(src2.at[pl.ds(wid * NB, NB)], src_v)
    pltpu.sync_copy(dst2.at[pl.ds(wid * NB, NB)], dst_v)
    pltpu.sync_copy(m16, m_v)
    mv = m_v[...]

    def _zb(j, _):
        z_r[j] = jnp.zeros((16,), _f32)
        return 0
    lax.fori_loop(0, 200, _zb, 0)

    @pl.when(s < 10)
    def _():
        for q in range(5):
            pltpu.sync_copy(z_r, s_acc.at[pl.ds(s * 1000 + q * 200, 200)])
    plsc.subcore_barrier()

    for b in range(NB):
        gbase = wid * TPE + b * 128
        pltpu.sync_copy(a_s.at[src_v.at[b]], as_r)
        pltpu.sync_copy(a_d.at[dst_v.at[b]], ad_r)

        def _wb(j, _):
            e = as_r[j] + ad_r[j]
            e = jnp.where(e >= 0.0, e, 0.2 * e)
            wv = jnp.exp(e - mv)
            valid = jnp.where(gbase + j < E + N, 1.0, 0.0).astype(_f32)
            w_r[j] = wv * valid
            return 0
        lax.fori_loop(0, 128, _wb, 0)
        pltpu.sync_copy(w_r, w_out.at[pl.ds(gbase, 128)])
        pltpu.sync_copy(w_r, s_acc.at[dst_v.at[b]], add=True)

    plsc.subcore_barrier()

    @pl.when(s < 10)
    def _():
        for q in range(5):
            r0 = s * 1000 + q * 200
            pltpu.sync_copy(s_acc.at[pl.ds(r0, 200)],
                            spart_out.at[c, pl.ds(r0, 200)])


def _b1(src2, dst2, a_s, a_d, m16):
    f = pl.kernel(
        _b1_body,
        out_type=[
            jax.ShapeDtypeStruct((EP, H), _f32),
            jax.ShapeDtypeStruct((NC, N, H), _f32),
        ],
        mesh=plsc.VectorSubcoreMesh(core_axis_name="c", subcore_axis_name="s"),
        compiler_params=pltpu.CompilerParams(
            use_tc_tiling_on_sc=False, needs_layout_passes=False),
        scratch_types=[
            pltpu.VMEM_SHARED((N, H), _f32),
            pltpu.VMEM((NB, 128), _i32),
            pltpu.VMEM((NB, 128), _i32),
            pltpu.VMEM((128, H), _f32),
            pltpu.VMEM((128, H), _f32),
            pltpu.VMEM((128, H), _f32),
            pltpu.VMEM((16,), _f32),
            pltpu.VMEM((200, H), _f32),
        ],
    )
    return f(src2, dst2, a_s, a_d, m16)


def _b2_body(dst2, w_in, s0, s1, alpha_out,
             dst_v, w_r, s0_r, s1_r, a_r):
    c = lax.axis_index("c")
    s = lax.axis_index("s")
    wid = s * NC + c
    pltpu.sync_copy(dst2.at[pl.ds(wid * NB, NB)], dst_v)
    for b in range(NB):
        gbase = wid * TPE + b * 128
        pltpu.sync_copy(w_in.at[pl.ds(gbase, 128)], w_r)
        pltpu.sync_copy(s0.at[dst_v.at[b]], s0_r)
        pltpu.sync_copy(s1.at[dst_v.at[b]], s1_r)

        def _ab(j, _):
            a_r[j] = w_r[j] / (s0_r[j] + s1_r[j] + 1e-16)
            return 0
        lax.fori_loop(0, 128, _ab, 0)
        pltpu.sync_copy(a_r, alpha_out.at[pl.ds(gbase, 128)])


def _b2(dst2, w, s0, s1):
    f = pl.kernel(
        _b2_body,
        out_type=[jax.ShapeDtypeStruct((EP, H), _f32)],
        mesh=plsc.VectorSubcoreMesh(core_axis_name="c", subcore_axis_name="s"),
        compiler_params=pltpu.CompilerParams(
            use_tc_tiling_on_sc=False, needs_layout_passes=False),
        scratch_types=[
            pltpu.VMEM((NB, 128), _i32),
            pltpu.VMEM((128, H), _f32),
            pltpu.VMEM((128, H), _f32),
            pltpu.VMEM((128, H), _f32),
            pltpu.VMEM((128, H), _f32),
        ],
    )
    return f(dst2, w, s0, s1)[0]


def _b3_body(src2, dst2, alpha, h, hidx, att_out,
             acc, src_v, dst_v, ids_v, hrowA, hrowB, arow2, zrow, hidx_v,
             nidx2, didx2, eidx2, gsem, asem, ssem):
    c = lax.axis_index("c")
    s = lax.axis_index("s")
    sbase = s * SEP
    pltpu.sync_copy(src2.at[pl.ds(s * (SEP // 128), SEP // 128)], src_v)
    pltpu.sync_copy(dst2.at[pl.ds(s * (SEP // 128), SEP // 128)], dst_v)
    pltpu.sync_copy(hidx, hidx_v)

    def _zj(j, _):
        def _zk(k, _):
            zrow[j, pl.ds(k * 16, 16)] = jnp.zeros((16,), _f32)
            return 0
        lax.fori_loop(0, C, _zk, 0)
        return 0
    lax.fori_loop(0, 8, _zj, 0)

    iota16 = lax.iota(_i32, 16)
    gdn = lax.GatherDimensionNumbers(
        offset_dims=(), collapsed_slice_dims=(0,), start_index_map=(0,))

    def _vgather(x, idx):
        return lax.gather(x, idx[:, None], gdn, (1,),
                          mode=lax.GatherScatterMode.PROMISE_IN_BOUNDS)

    shifts = [(jnp.maximum(iota16 - stp, 0), iota16 >= stp)
              for stp in (1, 2, 4, 8)]

    def _prefix_inclusive(v):
        for idx, sel in shifts:
            sh = jnp.where(sel, _vgather(v, idx), 0)
            v = v + sh
        return v

    def _zi(g, _):
        ids_v[pl.ds(g * 16, 16)] = jnp.zeros((16,), _i32)
        return 0
    lax.fori_loop(0, SGRP + 1, _zi, 0)

    def _chunk_body(chunk, _):
        cid = chunk * NC + c
        base = cid * CHUNK

        def _za(q, _):
            pltpu.sync_copy(zrow,
                            acc.at[pl.ds(s * ROWS_PER_TILE + q * 8, 8)])
            return 0
        lax.fori_loop(0, ROWS_PER_TILE // 8, _za, 0)
        plsc.subcore_barrier()

        def _scan(g, off):
            dv = dst_v[g >> 3, pl.ds((g & 7) * 16, 16)]
            mask = lax.shift_right_logical(dv, CSHIFT) == cid
            ids = iota16 + g * 16
            cnt = _prefix_inclusive(mask.astype(_i32))
            pos = off + cnt - 1
            plsc.store_scatter(ids_v, [pos], ids, mask=mask)
            return off + plsc.all_reduce_population_count(mask)[0]
        m = lax.fori_loop(0, SGRP, _scan, 0)

        nb = (m + BSZ - 1) // BSZ

        def _prep(bi, slot):
            for half in range(BSZ // 16):
                idsv = ids_v[pl.ds(bi * BSZ + half * 16, 16)]
                gr = lax.shift_right_logical(idsv, 7)
                gc = jnp.bitwise_and(idsv, 127)
                sv = plsc.load_gather(src_v, [gr, gc])
                dv = plsc.load_gather(dst_v, [gr, gc])
                nidx2[slot, pl.ds(half * 16, 16)] = sv
                didx2[slot, pl.ds(half * 16, 16)] = jnp.clip(
                    dv - base, 0, CHUNK - 1)
                eidx2[slot, pl.ds(half * 16, 16)] = idsv + sbase

            @pl.when(slot == 0)
            def _():
                pltpu.async_copy(h.at[nidx2.at[0]], hrowA, gsem)

            @pl.when(slot == 1)
            def _():
                pltpu.async_copy(h.at[nidx2.at[1]], hrowB, gsem)
            pltpu.async_copy(alpha.at[eidx2.at[slot]], arow2.at[slot], asem)

        @pl.when(nb > 0)
        def _():
            _prep(0, 0)

        def _batch(b, _):
            sl = jnp.bitwise_and(b, 1)
            nsl = 1 - sl

            @pl.when(jnp.logical_and(b >= 1, b + 1 < nb))
            def _():
                pltpu.make_async_copy(h.at[nidx2.at[0]], hrowA, ssem).wait()

            @pl.when(b + 1 < nb)
            def _():
                _prep(b + 1, nsl)

            pltpu.make_async_copy(h.at[nidx2.at[0]], hrowA, gsem).wait()
            pltpu.make_async_copy(alpha.at[eidx2.at[sl]],
                                  arow2.at[sl], asem).wait()

            def _do_compute(hrow_ref):
                for grp in range(BSZ // 16):
                    avs = [arow2[sl, grp * 16 + j] *
                           jnp.where(b * BSZ + grp * 16 + j < m,
                                     1.0, 0.0).astype(_f32)
                           for j in range(16)]

                    def _kb(k, _, _avs=avs, _g=grp):
                        hx = hidx_v[pl.ds(k * 16, 16)]
                        for j in range(16):
                            hv = hrow_ref[_g * 16 + j, pl.ds(k * 16, 16)]
                            hrow_ref[_g * 16 + j, pl.ds(k * 16, 16)] = (
                                hv * _vgather(_avs[j], hx))
                        return 0
                    lax.fori_loop(0, C, _kb, 0)

            @pl.when(sl == 0)
            def _():
                _do_compute(hrowA)
                pltpu.async_copy(hrowA, acc.at[didx2.at[0]], ssem, add=True)

            @pl.when(sl == 1)
            def _():
                _do_compute(hrowB)
                pltpu.async_copy(hrowB, acc.at[didx2.at[1]], ssem, add=True)
            return 0
        lax.fori_loop(0, nb, _batch, 0)

        def _drain(d, _):
            pltpu.make_async_copy(h.at[nidx2.at[0]], hrowA, ssem).wait()
            return 0
        lax.fori_loop(0, jnp.minimum(nb, 2), _drain, 0)
        plsc.subcore_barrier()

        pltpu.sync_copy(acc.at[pl.ds(s * ROWS_PER_TILE, ROWS_PER_TILE)],
                        att_out.at[pl.ds(base + s * ROWS_PER_TILE,
                                         ROWS_PER_TILE)])
        plsc.subcore_barrier()
        return 0

    lax.fori_loop(0, NCHUNKS // NC, _chunk_body, 0)


def _b3(src2, dstb2, alpha, h, hidx):
    f = pl.kernel(
        _b3_body,
        out_type=[jax.ShapeDtypeStruct((NCHUNKS * CHUNK, FEAT), _f32)],
        mesh=plsc.VectorSubcoreMesh(core_axis_name="c", subcore_axis_name="s"),
        compiler_params=pltpu.CompilerParams(
            use_tc_tiling_on_sc=False, needs_layout_passes=False),
        scratch_types=[
            pltpu.VMEM_SHARED((CHUNK, FEAT), _f32),
            pltpu.VMEM((SEP // 128, 128), _i32),
            pltpu.VMEM((SEP // 128, 128), _i32),
            pltpu.VMEM((SEP + 16,), _i32),
            pltpu.VMEM((BSZ, FEAT), _f32),
            pltpu.VMEM((BSZ, FEAT), _f32),
            pltpu.VMEM((2, BSZ, H), _f32),
            pltpu.VMEM((8, FEAT), _f32),
            pltpu.VMEM((FEAT,), _i32),
            pltpu.VMEM((2, BSZ), _i32),
            pltpu.VMEM((2, BSZ), _i32),
            pltpu.VMEM((2, BSZ), _i32),
            pltpu.SemaphoreType.DMA,
            pltpu.SemaphoreType.DMA,
            pltpu.SemaphoreType.DMA,
        ],
    )
    return f(src2, dstb2, alpha, h, hidx)[0]


def _c1_body(prev_ref, att_ref, bias_ref, s1_ref, s2_ref):
    i = pl.program_id(0)
    z = prev_ref[...] + att_ref[...] + bias_ref[...]
    s1 = jnp.sum(z, axis=0, keepdims=True)
    s2 = jnp.sum(z * z, axis=0, keepdims=True)

    @pl.when(i == 0)
    def _():
        s1_ref[...] = s1
        s2_ref[...] = s2

    @pl.when(i > 0)
    def _():
        s1_ref[...] += s1
        s2_ref[...] += s2


def _c2_body(prev_ref, att_ref, bias_ref, gamma_ref, beta_ref,
             s1_ref, s2_ref, y_ref):
    z = prev_ref[...] + att_ref[...] + bias_ref[...]
    mean = s1_ref[...] / N
    var = s2_ref[...] / N - mean * mean
    inv = lax.rsqrt(var + 1e-5)
    y = gamma_ref[...] * ((z - mean) * inv) + beta_ref[...]
    y_ref[...] = jnp.maximum(y, 0.0)


def _bn(prev, att, bias2, gamma2, beta2):
    s1, s2 = pl.pallas_call(
        _c1_body,
        grid=(GRID,),
        in_specs=[
            pl.BlockSpec((MBLK, FEAT), lambda i: (i, 0)),
            pl.BlockSpec((MBLK, FEAT), lambda i: (i, 0)),
            pl.BlockSpec((1, FEAT), lambda i: (0, 0)),
        ],
        out_specs=[
            pl.BlockSpec((1, FEAT), lambda i: (0, 0)),
            pl.BlockSpec((1, FEAT), lambda i: (0, 0)),
        ],
        out_shape=[
            jax.ShapeDtypeStruct((1, FEAT), _f32),
            jax.ShapeDtypeStruct((1, FEAT), _f32),
        ],
    )(prev, att, bias2)
    return pl.pallas_call(
        _c2_body,
        grid=(GRID,),
        in_specs=[
            pl.BlockSpec((MBLK, FEAT), lambda i: (i, 0)),
            pl.BlockSpec((MBLK, FEAT), lambda i: (i, 0)),
            pl.BlockSpec((1, FEAT), lambda i: (0, 0)),
            pl.BlockSpec((1, FEAT), lambda i: (0, 0)),
            pl.BlockSpec((1, FEAT), lambda i: (0, 0)),
            pl.BlockSpec((1, FEAT), lambda i: (0, 0)),
            pl.BlockSpec((1, FEAT), lambda i: (0, 0)),
        ],
        out_specs=pl.BlockSpec((MBLK, FEAT), lambda i: (i, 0)),
        out_shape=jax.ShapeDtypeStruct((N, FEAT), _f32),
    )(prev, att, bias2, gamma2, beta2, s1, s2)


def kernel(prev, x, edge_index, W, att_src, att_dst, bias, gamma, beta):
    loops = jnp.arange(N, dtype=_i32)
    pad = jnp.zeros((EP - E - N,), dtype=_i32)
    src = jnp.concatenate([edge_index[0], loops, pad])
    dst = jnp.concatenate([edge_index[1], loops, pad])
    dstb = jnp.concatenate(
        [edge_index[1], loops, jnp.full((EP - E - N,), DST_SENTINEL, _i32)])
    src2 = src.reshape(EP // 128, 128)
    dst2 = dst.reshape(EP // 128, 128)
    dstb2 = dstb.reshape(EP // 128, 128)

    rows = jnp.arange(FEAT)
    S_src = jnp.zeros((FEAT, H), _f32).at[rows, rows // C].set(
        att_src.reshape(-1))
    S_dst = jnp.zeros((FEAT, H), _f32).at[rows, rows // C].set(
        att_dst.reshape(-1))

    h, a_s, a_d, ms, md = _project(x, W, S_src, S_dst)

    v = ms[0, 0] + md[0, 0]
    M = jnp.where(v >= 0.0, v, 0.2 * v)
    m16 = jnp.full((16,), M, _f32)

    w, spart = _b1(src2, dst2, a_s, a_d, m16)
    alpha = _b2(dst2, w, spart[0], spart[1])

    hidx = (rows // C).astype(_i32)
    att = _b3(src2, dstb2, alpha, h, hidx)

    bias2 = bias.reshape(1, FEAT)
    gamma2 = gamma.reshape(1, FEAT)
    beta2 = beta.reshape(1, FEAT)
    return _bn(prev, att, bias2, gamma2, beta2)


# speedup vs baseline: 2.5398x; 1.0786x over previous
import functools

import jax
import jax.numpy as jnp
from jax import lax
from jax.experimental import pallas as pl
from jax.experimental.pallas import tpu as pltpu
from jax.experimental.pallas import tpu_sc as plsc

N = 10000
FEAT = 1040
H = 16
C = 65
E = 76800

NC = 2
NS = 16
EP = 98304
TPE = EP // (NC * NS)
NB = TPE // 128
SEP = EP // NS
SGRP = SEP // 16
DST_SENTINEL = 16000
CHUNK = 512
CSHIFT = 9
NCHUNKS = 20
ROWS_PER_TILE = CHUNK // NS
BSZ = 32
MBLK = 400
GRID = N // MBLK

_f32 = jnp.float32
_i32 = jnp.int32


def _proj_body(x_ref, w_ref, ss_ref, sd_ref, h_ref, as_ref, ad_ref,
               ms_ref, md_ref):
    i = pl.program_id(0)
    h = jnp.dot(x_ref[...], w_ref[...], preferred_element_type=_f32)
    h_ref[...] = h
    a_s = jnp.dot(h, ss_ref[...], preferred_element_type=_f32)
    a_d = jnp.dot(h, sd_ref[...], preferred_element_type=_f32)
    as_ref[...] = a_s
    ad_ref[...] = a_d
    bms = jnp.max(a_s)
    bmd = jnp.max(a_d)

    @pl.when(i == 0)
    def _():
        ms_ref[0, 0] = bms
        md_ref[0, 0] = bmd

    @pl.when(i > 0)
    def _():
        ms_ref[0, 0] = jnp.maximum(ms_ref[0, 0], bms)
        md_ref[0, 0] = jnp.maximum(md_ref[0, 0], bmd)


def _project(x, W, S_src, S_dst):
    return pl.pallas_call(
        _proj_body,
        grid=(GRID,),
        in_specs=[
            pl.BlockSpec((MBLK, FEAT), lambda i: (i, 0)),
            pl.BlockSpec((FEAT, FEAT), lambda i: (0, 0)),
            pl.BlockSpec((FEAT, H), lambda i: (0, 0)),
            pl.BlockSpec((FEAT, H), lambda i: (0, 0)),
        ],
        out_specs=[
            pl.BlockSpec((MBLK, FEAT), lambda i: (i, 0)),
            pl.BlockSpec((MBLK, H), lambda i: (i, 0)),
            pl.BlockSpec((MBLK, H), lambda i: (i, 0)),
            pl.BlockSpec((1, 1), lambda i: (0, 0), memory_space=pltpu.SMEM),
            pl.BlockSpec((1, 1), lambda i: (0, 0), memory_space=pltpu.SMEM),
        ],
        out_shape=[
            jax.ShapeDtypeStruct((N, FEAT), _f32),
            jax.ShapeDtypeStruct((N, H), _f32),
            jax.ShapeDtypeStruct((N, H), _f32),
            jax.ShapeDtypeStruct((1, 1), _f32),
            jax.ShapeDtypeStruct((1, 1), _f32),
        ],
    )(x, W, S_src, S_dst)


def _b1_body(src2, dst2, a_s, a_d, m16, w_out, spart_out,
             s_acc, src_v, dst_v, as_r, ad_r, w_r, m_v, z_r):
    c = lax.axis_index("c")
    s = lax.axis_index("s")
    wid = s * NC + c
    pltpu.sync_copy(src2.at[pl.ds(wid * NB, NB)], src_v)
    pltpu.sync_copy(dst2.at[pl.ds(wid * NB, NB)], dst_v)
    pltpu.sync_copy(m16, m_v)
    mv = m_v[...]

    def _zb(j, _):
        z_r[j] = jnp.zeros((16,), _f32)
        return 0
    lax.fori_loop(0, 200, _zb, 0)

    @pl.when(s < 10)
    def _():
        for q in range(5):
            pltpu.sync_copy(z_r, s_acc.at[pl.ds(s * 1000 + q * 200, 200)])
    plsc.subcore_barrier()

    for b in range(NB):
        gbase = wid * TPE + b * 128
        pltpu.sync_copy(a_s.at[src_v.at[b]], as_r)
        pltpu.sync_copy(a_d.at[dst_v.at[b]], ad_r)

        def _wb(j, _):
            e = as_r[j] + ad_r[j]
            e = jnp.where(e >= 0.0, e, 0.2 * e)
            wv = jnp.exp(e - mv)
            valid = jnp.where(gbase + j < E + N, 1.0, 0.0).astype(_f32)
            w_r[j] = wv * valid
            return 0
        lax.fori_loop(0, 128, _wb, 0)
        pltpu.sync_copy(w_r, w_out.at[pl.ds(gbase, 128)])
        pltpu.sync_copy(w_r, s_acc.at[dst_v.at[b]], add=True)

    plsc.subcore_barrier()

    @pl.when(s < 10)
    def _():
        for q in range(5):
            r0 = s * 1000 + q * 200
            pltpu.sync_copy(s_acc.at[pl.ds(r0, 200)],
                            spart_out.at[c, pl.ds(r0, 200)])


def _b1(src2, dst2, a_s, a_d, m16):
    f = pl.kernel(
        _b1_body,
        out_type=[
            jax.ShapeDtypeStruct((EP, H), _f32),
            jax.ShapeDtypeStruct((NC, N, H), _f32),
        ],
        mesh=plsc.VectorSubcoreMesh(core_axis_name="c", subcore_axis_name="s"),
        compiler_params=pltpu.CompilerParams(
            use_tc_tiling_on_sc=False, needs_layout_passes=False),
        scratch_types=[
            pltpu.VMEM_SHARED((N, H), _f32),
            pltpu.VMEM((NB, 128), _i32),
            pltpu.VMEM((NB, 128), _i32),
            pltpu.VMEM((128, H), _f32),
            pltpu.VMEM((128, H), _f32),
            pltpu.VMEM((128, H), _f32),
            pltpu.VMEM((16,), _f32),
            pltpu.VMEM((200, H), _f32),
        ],
    )
    return f(src2, dst2, a_s, a_d, m16)


def _b2_body(dst2, w_in, s0, s1, alpha_out,
             dst_v, w_r, s0_r, s1_r, a_r):
    c = lax.axis_index("c")
    s = lax.axis_index("s")
    wid = s * NC + c
    pltpu.sync_copy(dst2.at[pl.ds(wid * NB, NB)], dst_v)
    for b in range(NB):
        gbase = wid * TPE + b * 128
        pltpu.sync_copy(w_in.at[pl.ds(gbase, 128)], w_r)
        pltpu.sync_copy(s0.at[dst_v.at[b]], s0_r)
        pltpu.sync_copy(s1.at[dst_v.at[b]], s1_r)

        def _ab(j, _):
            a_r[j] = w_r[j] / (s0_r[j] + s1_r[j] + 1e-16)
            return 0
        lax.fori_loop(0, 128, _ab, 0)
        pltpu.sync_copy(a_r, alpha_out.at[pl.ds(gbase, 128)])


def _b2(dst2, w, s0, s1):
    f = pl.kernel(
        _b2_body,
        out_type=[jax.ShapeDtypeStruct((EP, H), _f32)],
        mesh=plsc.VectorSubcoreMesh(core_axis_name="c", subcore_axis_name="s"),
        compiler_params=pltpu.CompilerParams(
            use_tc_tiling_on_sc=False, needs_layout_passes=False),
        scratch_types=[
            pltpu.VMEM((NB, 128), _i32),
            pltpu.VMEM((128, H), _f32),
            pltpu.VMEM((128, H), _f32),
            pltpu.VMEM((128, H), _f32),
            pltpu.VMEM((128, H), _f32),
        ],
    )
    return f(dst2, w, s0, s1)[0]


def _b3_body(src2, dst2, w_in, s0, s1, h, hidx, att_out,
             acc, src_v, dst_v, ids_v, hrowA, hrowB, wrow2, s0row2, s1row2,
             zrow, hidx_v, nidx2, didx2, eidx2, dstid2, gsem, asem, ssem):
    c = lax.axis_index("c")
    s = lax.axis_index("s")
    sbase = s * SEP
    pltpu.sync_copy(src2.at[pl.ds(s * (SEP // 128), SEP // 128)], src_v)
    pltpu.sync_copy(dst2.at[pl.ds(s * (SEP // 128), SEP // 128)], dst_v)
    pltpu.sync_copy(hidx, hidx_v)

    def _zj(j, _):
        def _zk(k, _):
            zrow[j, pl.ds(k * 16, 16)] = jnp.zeros((16,), _f32)
            return 0
        lax.fori_loop(0, C, _zk, 0)
        return 0
    lax.fori_loop(0, 8, _zj, 0)

    iota16 = lax.iota(_i32, 16)
    gdn = lax.GatherDimensionNumbers(
        offset_dims=(), collapsed_slice_dims=(0,), start_index_map=(0,))

    def _vgather(x, idx):
        return lax.gather(x, idx[:, None], gdn, (1,),
                          mode=lax.GatherScatterMode.PROMISE_IN_BOUNDS)

    shifts = [(jnp.maximum(iota16 - stp, 0), iota16 >= stp)
              for stp in (1, 2, 4, 8)]

    def _prefix_inclusive(v):
        for idx, sel in shifts:
            sh = jnp.where(sel, _vgather(v, idx), 0)
            v = v + sh
        return v

    def _zi(g, _):
        ids_v[pl.ds(g * 16, 16)] = jnp.zeros((16,), _i32)
        return 0
    lax.fori_loop(0, SGRP + 1, _zi, 0)

    def _chunk_body(chunk, _):
        cid = chunk * NC + c
        base = cid * CHUNK

        def _za(q, _):
            pltpu.sync_copy(zrow,
                            acc.at[pl.ds(s * ROWS_PER_TILE + q * 8, 8)])
            return 0
        lax.fori_loop(0, ROWS_PER_TILE // 8, _za, 0)
        plsc.subcore_barrier()

        def _scan(g, off):
            dv = dst_v[g >> 3, pl.ds((g & 7) * 16, 16)]
            mask = lax.shift_right_logical(dv, CSHIFT) == cid
            ids = iota16 + g * 16
            cnt = _prefix_inclusive(mask.astype(_i32))
            pos = off + cnt - 1
            plsc.store_scatter(ids_v, [pos], ids, mask=mask)
            return off + plsc.all_reduce_population_count(mask)[0]
        m = lax.fori_loop(0, SGRP, _scan, 0)

        nb = (m + BSZ - 1) // BSZ

        def _prep(bi, slot):
            for half in range(BSZ // 16):
                idsv = ids_v[pl.ds(bi * BSZ + half * 16, 16)]
                gr = lax.shift_right_logical(idsv, 7)
                gc = jnp.bitwise_and(idsv, 127)
                sv = plsc.load_gather(src_v, [gr, gc])
                dv = plsc.load_gather(dst_v, [gr, gc])
                nidx2[slot, pl.ds(half * 16, 16)] = sv
                didx2[slot, pl.ds(half * 16, 16)] = jnp.clip(
                    dv - base, 0, CHUNK - 1)
                eidx2[slot, pl.ds(half * 16, 16)] = idsv + sbase
                dstid2[slot, pl.ds(half * 16, 16)] = dv

            @pl.when(slot == 0)
            def _():
                pltpu.async_copy(h.at[nidx2.at[0]], hrowA, gsem)

            @pl.when(slot == 1)
            def _():
                pltpu.async_copy(h.at[nidx2.at[1]], hrowB, gsem)
            pltpu.async_copy(w_in.at[eidx2.at[slot]], wrow2.at[slot], asem)
            pltpu.async_copy(s0.at[dstid2.at[slot]], s0row2.at[slot], asem)
            pltpu.async_copy(s1.at[dstid2.at[slot]], s1row2.at[slot], asem)

        @pl.when(nb > 0)
        def _():
            _prep(0, 0)

        def _batch(b, _):
            sl = jnp.bitwise_and(b, 1)
            nsl = 1 - sl

            @pl.when(jnp.logical_and(b >= 1, b + 1 < nb))
            def _():
                pltpu.make_async_copy(h.at[nidx2.at[0]], hrowA, ssem).wait()

            @pl.when(b + 1 < nb)
            def _():
                _prep(b + 1, nsl)

            pltpu.make_async_copy(h.at[nidx2.at[0]], hrowA, gsem).wait()
            pltpu.make_async_copy(w_in.at[eidx2.at[sl]],
                                  wrow2.at[sl], asem).wait()
            pltpu.make_async_copy(s0.at[dstid2.at[sl]],
                                  s0row2.at[sl], asem).wait()
            pltpu.make_async_copy(s1.at[dstid2.at[sl]],
                                  s1row2.at[sl], asem).wait()

            def _do_compute(hrow_ref):
                for grp in range(BSZ // 16):
                    avs = [wrow2[sl, grp * 16 + j] /
                           (s0row2[sl, grp * 16 + j] +
                            s1row2[sl, grp * 16 + j] + 1e-16) *
                           jnp.where(b * BSZ + grp * 16 + j < m,
                                     1.0, 0.0).astype(_f32)
                           for j in range(16)]

                    def _kb(k, _, _avs=avs, _g=grp):
                        hx = hidx_v[pl.ds(k * 16, 16)]
                        for j in range(16):
                            hv = hrow_ref[_g * 16 + j, pl.ds(k * 16, 16)]
                            hrow_ref[_g * 16 + j, pl.ds(k * 16, 16)] = (
                                hv * _vgather(_avs[j], hx))
                        return 0
                    lax.fori_loop(0, C, _kb, 0)

            @pl.when(sl == 0)
            def _():
                _do_compute(hrowA)
                pltpu.async_copy(hrowA, acc.at[didx2.at[0]], ssem, add=True)

            @pl.when(sl == 1)
            def _():
                _do_compute(hrowB)
                pltpu.async_copy(hrowB, acc.at[didx2.at[1]], ssem, add=True)
            return 0
        lax.fori_loop(0, nb, _batch, 0)

        def _drain(d, _):
            pltpu.make_async_copy(h.at[nidx2.at[0]], hrowA, ssem).wait()
            return 0
        lax.fori_loop(0, jnp.minimum(nb, 2), _drain, 0)
        plsc.subcore_barrier()

        pltpu.sync_copy(acc.at[pl.ds(s * ROWS_PER_TILE, ROWS_PER_TILE)],
                        att_out.at[pl.ds(base + s * ROWS_PER_TILE,
                                         ROWS_PER_TILE)])
        plsc.subcore_barrier()
        return 0

    lax.fori_loop(0, NCHUNKS // NC, _chunk_body, 0)


def _b3(src2, dstb2, w, s0, s1, h, hidx):
    f = pl.kernel(
        _b3_body,
        out_type=[jax.ShapeDtypeStruct((NCHUNKS * CHUNK, FEAT), _f32)],
        mesh=plsc.VectorSubcoreMesh(core_axis_name="c", subcore_axis_name="s"),
        compiler_params=pltpu.CompilerParams(
            use_tc_tiling_on_sc=False, needs_layout_passes=False),
        scratch_types=[
            pltpu.VMEM_SHARED((CHUNK, FEAT), _f32),
            pltpu.VMEM((SEP // 128, 128), _i32),
            pltpu.VMEM((SEP // 128, 128), _i32),
            pltpu.VMEM((SEP + 16,), _i32),
            pltpu.VMEM((BSZ, FEAT), _f32),
            pltpu.VMEM((BSZ, FEAT), _f32),
            pltpu.VMEM((2, BSZ, H), _f32),
            pltpu.VMEM((2, BSZ, H), _f32),
            pltpu.VMEM((2, BSZ, H), _f32),
            pltpu.VMEM((8, FEAT), _f32),
            pltpu.VMEM((FEAT,), _i32),
            pltpu.VMEM((2, BSZ), _i32),
            pltpu.VMEM((2, BSZ), _i32),
            pltpu.VMEM((2, BSZ), _i32),
            pltpu.VMEM((2, BSZ), _i32),
            pltpu.SemaphoreType.DMA,
            pltpu.SemaphoreType.DMA,
            pltpu.SemaphoreType.DMA,
        ],
    )
    return f(src2, dstb2, w, s0, s1, h, hidx)[0]


def _c1_body(prev_ref, att_ref, bias_ref, s1_ref, s2_ref):
    i = pl.program_id(0)
    z = prev_ref[...] + att_ref[...] + bias_ref[...]
    s1 = jnp.sum(z, axis=0, keepdims=True)
    s2 = jnp.sum(z * z, axis=0, keepdims=True)

    @pl.when(i == 0)
    def _():
        s1_ref[...] = s1
        s2_ref[...] = s2

    @pl.when(i > 0)
    def _():
        s1_ref[...] += s1
        s2_ref[...] += s2


def _c2_body(prev_ref, att_ref, bias_ref, gamma_ref, beta_ref,
             s1_ref, s2_ref, y_ref):
    z = prev_ref[...] + att_ref[...] + bias_ref[...]
    mean = s1_ref[...] / N
    var = s2_ref[...] / N - mean * mean
    inv = lax.rsqrt(var + 1e-5)
    y = gamma_ref[...] * ((z - mean) * inv) + beta_ref[...]
    y_ref[...] = jnp.maximum(y, 0.0)


def _bn(prev, att, bias2, gamma2, beta2):
    s1, s2 = pl.pallas_call(
        _c1_body,
        grid=(GRID,),
        in_specs=[
            pl.BlockSpec((MBLK, FEAT), lambda i: (i, 0)),
            pl.BlockSpec((MBLK, FEAT), lambda i: (i, 0)),
            pl.BlockSpec((1, FEAT), lambda i: (0, 0)),
        ],
        out_specs=[
            pl.BlockSpec((1, FEAT), lambda i: (0, 0)),
            pl.BlockSpec((1, FEAT), lambda i: (0, 0)),
        ],
        out_shape=[
            jax.ShapeDtypeStruct((1, FEAT), _f32),
            jax.ShapeDtypeStruct((1, FEAT), _f32),
        ],
    )(prev, att, bias2)
    return pl.pallas_call(
        _c2_body,
        grid=(GRID,),
        in_specs=[
            pl.BlockSpec((MBLK, FEAT), lambda i: (i, 0)),
            pl.BlockSpec((MBLK, FEAT), lambda i: (i, 0)),
            pl.BlockSpec((1, FEAT), lambda i: (0, 0)),
            pl.BlockSpec((1, FEAT), lambda i: (0, 0)),
            pl.BlockSpec((1, FEAT), lambda i: (0, 0)),
            pl.BlockSpec((1, FEAT), lambda i: (0, 0)),
            pl.BlockSpec((1, FEAT), lambda i: (0, 0)),
        ],
        out_specs=pl.BlockSpec((MBLK, FEAT), lambda i: (i, 0)),
        out_shape=jax.ShapeDtypeStruct((N, FEAT), _f32),
    )(prev, att, bias2, gamma2, beta2, s1, s2)


def kernel(prev, x, edge_index, W, att_src, att_dst, bias, gamma, beta):
    loops = jnp.arange(N, dtype=_i32)
    pad = jnp.zeros((EP - E - N,), dtype=_i32)
    src = jnp.concatenate([edge_index[0], loops, pad])
    dst = jnp.concatenate([edge_index[1], loops, pad])
    dstb = jnp.concatenate(
        [edge_index[1], loops, jnp.full((EP - E - N,), DST_SENTINEL, _i32)])
    src2 = src.reshape(EP // 128, 128)
    dst2 = dst.reshape(EP // 128, 128)
    dstb2 = dstb.reshape(EP // 128, 128)

    rows = jnp.arange(FEAT)
    S_src = jnp.zeros((FEAT, H), _f32).at[rows, rows // C].set(
        att_src.reshape(-1))
    S_dst = jnp.zeros((FEAT, H), _f32).at[rows, rows // C].set(
        att_dst.reshape(-1))

    h, a_s, a_d, ms, md = _project(x, W, S_src, S_dst)

    v = ms[0, 0] + md[0, 0]
    M = jnp.where(v >= 0.0, v, 0.2 * v)
    m16 = jnp.full((16,), M, _f32)

    w, spart = _b1(src2, dst2, a_s, a_d, m16)

    hidx = (rows // C).astype(_i32)
    att = _b3(src2, dstb2, w, spart[0], spart[1], h, hidx)

    bias2 = bias.reshape(1, FEAT)
    gamma2 = gamma.reshape(1, FEAT)
    beta2 = beta.reshape(1, FEAT)
    return _bn(prev, att, bias2, gamma2, beta2)
